# baseline (speedup 1.0000x reference)
import functools

import jax
import jax.numpy as jnp
from jax import lax
from jax.experimental import pallas as pl
from jax.experimental.pallas import tpu as pltpu
from jax.experimental.pallas import tpu_sc as plsc

_NC = 2
_NS = 16
_NW = _NC * _NS
_L = 16

_C1 = 1.42459928987172
_C2 = -0.5892300281498137
_C3 = 0.16540458521111767

_MANT_MASK = 0x7FFFFF
_ONE_BITS = 0x3F800000


def _sc_body(nchunks, chunk, hard_hbm, soft_hbm, out_f, out_e, out_c, out_z,
             hard_buf, soft_buf, stage_f, stage_e, stage_c, stage_z,
             sem0, sem1):
    wid = lax.axis_index("c") * _NS + lax.axis_index("s")
    per_w = nchunks * chunk
    base = wid * per_w
    sems = (sem0, sem1)

    def _start(g, b):
        pltpu.async_copy(hard_hbm.at[pl.ds(base + g * chunk, chunk)],
                         hard_buf.at[b], sems[b])
        pltpu.async_copy(soft_hbm.at[pl.ds(base + g * chunk, chunk)],
                         soft_buf.at[b], sems[b])

    def _wait(b):
        pltpu.make_async_copy(hard_hbm.at[pl.ds(0, chunk)],
                              hard_buf.at[b], sems[b]).wait()
        pltpu.make_async_copy(soft_hbm.at[pl.ds(0, chunk)],
                              soft_buf.at[b], sems[b]).wait()

    _start(0, 0)
    _start(1, 1)

    niter = chunk // _L

    def _chunk_acc(b, accf, acce, cnt, zcnt):
        hb = hard_buf.at[b]
        sb_ = soft_buf.at[b]

        def _iter(j, carry):
            af, ae, ac, az = carry
            h = hb[pl.ds(j * _L, _L)]
            sbits = lax.bitcast_convert_type(sb_[pl.ds(j * _L, _L)], jnp.int32)
            sbm = sbits & (-h)
            e = lax.shift_right_logical(sbm, 23)
            mbits = (sbm & _MANT_MASK) | _ONE_BITS
            t = lax.bitcast_convert_type(mbits, jnp.float32) - 1.0
            p = t * (_C1 + t * (_C2 + t * _C3))
            z = lax.shift_right_logical(sbm - h, 31)
            return af + p, ae + e, ac + h, az + z

        return lax.fori_loop(0, niter, _iter, (accf, acce, cnt, zcnt),
                             unroll=8)

    def _outer(gg, carry):
        accf, acce, cnt, zcnt = carry
        for b in (0, 1):
            g = gg * 2 + b
            _wait(b)
            accf, acce, cnt, zcnt = _chunk_acc(b, accf, acce, cnt, zcnt)

            @pl.when(g + 2 < nchunks)
            def _():
                _start(g + 2, b)
        return accf, acce, cnt, zcnt

    zf = jnp.zeros((_L,), jnp.float32)
    zi = jnp.zeros((_L,), jnp.int32)
    accf, acce, cnt, zcnt = lax.fori_loop(0, nchunks // 2, _outer,
                                          (zf, zi, zi, zi))

    stage_f[...] = accf
    stage_e[...] = acce
    stage_c[...] = cnt
    stage_z[...] = zcnt
    pltpu.sync_copy(stage_f, out_f.at[wid])
    pltpu.sync_copy(stage_e, out_e.at[wid])
    pltpu.sync_copy(stage_c, out_c.at[wid])
    pltpu.sync_copy(stage_z, out_z.at[wid])


def kernel(hard_attention, soft_attention):
    n = hard_attention.size
    per_w = n // _NW
    chunk = 16384
    nchunks = per_w // chunk

    mesh = plsc.VectorSubcoreMesh(core_axis_name="c", subcore_axis_name="s")
    sc = pl.kernel(
        functools.partial(_sc_body, nchunks, chunk),
        out_type=[
            jax.ShapeDtypeStruct((_NW, _L), jnp.float32),
            jax.ShapeDtypeStruct((_NW, _L), jnp.int32),
            jax.ShapeDtypeStruct((_NW, _L), jnp.int32),
            jax.ShapeDtypeStruct((_NW, _L), jnp.int32),
        ],
        mesh=mesh,
        scratch_types=[
            pltpu.VMEM((2, chunk), jnp.int32),
            pltpu.VMEM((2, chunk), jnp.float32),
            pltpu.VMEM((_L,), jnp.float32),
            pltpu.VMEM((_L,), jnp.int32),
            pltpu.VMEM((_L,), jnp.int32),
            pltpu.VMEM((_L,), jnp.int32),
            pltpu.SemaphoreType.DMA,
            pltpu.SemaphoreType.DMA,
        ],
    )
    out_f, out_e, out_c, out_z = sc(hard_attention.reshape(-1),
                                    soft_attention.reshape(-1))

    sum_f = jnp.sum(out_f)
    sum_e = jnp.sum(out_e.astype(jnp.float32))
    sum_c = jnp.sum(out_c.astype(jnp.float32))
    any_zero = jnp.sum(out_z) > 0
    ln2 = 0.6931471805599453
    log_sum = ln2 * (sum_e - 127.0 * sum_c + sum_f)
    log_sum = jnp.where(any_zero, -jnp.inf, log_sum)
    return -log_sum / sum_c


# speedup vs baseline: 2.0303x; 2.0303x over previous
import functools

import jax
import jax.numpy as jnp
from jax import lax
from jax.experimental import pallas as pl
from jax.experimental.pallas import tpu as pltpu
from jax.experimental.pallas import tpu_sc as plsc

_NC = 2
_NS = 16
_NW = _NC * _NS
_L = 16
_RB = 8
_D = 2048

_C1 = 1.42459928987172
_C2 = -0.5892300281498137
_C3 = 0.16540458521111767

_MANT_MASK = 0x7FFFFF
_ONE_BITS = 0x3F800000


def _sc_body(nchunks, hard_hbm, soft_hbm, out_f, out_e, out_c, out_z,
             hard_buf, soft_buf, stage_f, stage_e, stage_c, stage_z,
             sem0, sem1):
    wid = lax.axis_index("c") * _NS + lax.axis_index("s")
    row_base = wid * (nchunks * _RB)
    sems = (sem0, sem1)

    def _start(g, b):
        r0 = row_base + g * _RB
        pltpu.async_copy(hard_hbm.at[pl.ds(r0, _RB)], hard_buf.at[b], sems[b])
        pltpu.async_copy(soft_hbm.at[pl.ds(r0, _RB)], soft_buf.at[b], sems[b])

    def _wait(b):
        pltpu.make_async_copy(hard_hbm.at[pl.ds(0, _RB)],
                              hard_buf.at[b], sems[b]).wait()
        pltpu.make_async_copy(soft_hbm.at[pl.ds(0, _RB)],
                              soft_buf.at[b], sems[b]).wait()

    _start(0, 0)
    _start(1, 1)

    def _chunk_acc(b, carry):
        hb = hard_buf.at[b]
        sb_ = soft_buf.at[b]

        def _iter(j, carry):
            af, ae, ac, az = carry
            for r in range(_RB):
                h = hb[r, pl.ds(j * _L, _L)]
                sbits = lax.bitcast_convert_type(
                    sb_[r, pl.ds(j * _L, _L)], jnp.int32)
                sbm = sbits * h
                e = lax.shift_right_logical(sbm, 23)
                mbits = (sbm & _MANT_MASK) | _ONE_BITS
                t = lax.bitcast_convert_type(mbits, jnp.float32) - 1.0
                p = t * (_C1 + t * (_C2 + t * _C3))
                z = lax.shift_right_logical(sbm - h, 31)
                af, ae, ac, az = af + p, ae + e, ac + h, az + z
            return af, ae, ac, az

        return lax.fori_loop(0, _D // _L, _iter, carry, unroll=2)

    def _outer(gg, carry):
        for b in (0, 1):
            g = gg * 2 + b
            _wait(b)
            carry = _chunk_acc(b, carry)

            @pl.when(g + 2 < nchunks)
            def _():
                _start(g + 2, b)
        return carry

    zf = jnp.zeros((_L,), jnp.float32)
    zi = jnp.zeros((_L,), jnp.int32)
    accf, acce, cnt, zcnt = lax.fori_loop(0, nchunks // 2, _outer,
                                          (zf, zi, zi, zi))

    stage_f[...] = accf
    stage_e[...] = acce
    stage_c[...] = cnt
    stage_z[...] = zcnt
    pltpu.sync_copy(stage_f, out_f.at[wid])
    pltpu.sync_copy(stage_e, out_e.at[wid])
    pltpu.sync_copy(stage_c, out_c.at[wid])
    pltpu.sync_copy(stage_z, out_z.at[wid])


def kernel(hard_attention, soft_attention):
    b, s, d = hard_attention.shape
    rows = b * s
    hard2 = hard_attention.reshape(rows, d)
    soft2 = soft_attention.reshape(rows, d)
    nchunks = rows // (_NW * _RB)

    mesh = plsc.VectorSubcoreMesh(core_axis_name="c", subcore_axis_name="s")
    sc = pl.kernel(
        functools.partial(_sc_body, nchunks),
        out_type=[
            jax.ShapeDtypeStruct((_NW, _L), jnp.float32),
            jax.ShapeDtypeStruct((_NW, _L), jnp.int32),
            jax.ShapeDtypeStruct((_NW, _L), jnp.int32),
            jax.ShapeDtypeStruct((_NW, _L), jnp.int32),
        ],
        mesh=mesh,
        compiler_params=pltpu.CompilerParams(use_tc_tiling_on_sc=True),
        scratch_types=[
            pltpu.VMEM((2, _RB, _D), jnp.int32),
            pltpu.VMEM((2, _RB, _D), jnp.float32),
            pltpu.VMEM((_L,), jnp.float32),
            pltpu.VMEM((_L,), jnp.int32),
            pltpu.VMEM((_L,), jnp.int32),
            pltpu.VMEM((_L,), jnp.int32),
            pltpu.SemaphoreType.DMA,
            pltpu.SemaphoreType.DMA,
        ],
    )
    out_f, out_e, out_c, out_z = sc(hard2, soft2)

    sum_f = jnp.sum(out_f)
    sum_e = jnp.sum(out_e.astype(jnp.float32))
    sum_c = jnp.sum(out_c.astype(jnp.float32))
    any_zero = jnp.sum(out_z) > 0
    ln2 = 0.6931471805599453
    log_sum = ln2 * (sum_e - 127.0 * sum_c + sum_f)
    log_sum = jnp.where(any_zero, -jnp.inf, log_sum)
    return -log_sum / sum_c


# speedup vs baseline: 4.0975x; 2.0181x over previous
import functools

import jax
import jax.numpy as jnp
from jax import lax
from jax.experimental import pallas as pl
from jax.experimental.pallas import tpu as pltpu
from jax.experimental.pallas import tpu_sc as plsc

_NC = 2
_NS = 16
_NW = _NC * _NS
_L = 16
_RB = 8
_D = 2048

_SC_BATCHES = 4

_C1 = 1.42459928987172
_C2 = -0.5892300281498137
_C3 = 0.16540458521111767

_MANT_MASK = 0x7FFFFF
_ONE_BITS = 0x3F800000


def _sc_body(sc_row0, nchunks, hard_hbm, soft_hbm, out_f, out_e, out_c,
             out_z, hard_buf, soft_buf, stage_f, stage_e, stage_c, stage_z,
             sem0, sem1):
    wid = lax.axis_index("c") * _NS + lax.axis_index("s")
    row_base = sc_row0 + wid * (nchunks * _RB)
    sems = (sem0, sem1)

    def _start(g, b):
        r0 = row_base + g * _RB
        pltpu.async_copy(hard_hbm.at[pl.ds(r0, _RB)], hard_buf.at[b], sems[b])
        pltpu.async_copy(soft_hbm.at[pl.ds(r0, _RB)], soft_buf.at[b], sems[b])

    def _wait(b):
        pltpu.make_async_copy(hard_hbm.at[pl.ds(0, _RB)],
                              hard_buf.at[b], sems[b]).wait()
        pltpu.make_async_copy(soft_hbm.at[pl.ds(0, _RB)],
                              soft_buf.at[b], sems[b]).wait()

    _start(0, 0)
    _start(1, 1)

    def _chunk_acc(b, carry):
        hb = hard_buf.at[b]
        sb_ = soft_buf.at[b]

        def _iter(j, carry):
            af, ae, ac, az = carry
            for r in range(_RB):
                h = hb[r, pl.ds(j * _L, _L)]
                sbits = lax.bitcast_convert_type(
                    sb_[r, pl.ds(j * _L, _L)], jnp.int32)
                sbm = sbits * h
                e = lax.shift_right_logical(sbm, 23)
                mbits = (sbm & _MANT_MASK) | _ONE_BITS
                t = lax.bitcast_convert_type(mbits, jnp.float32) - 1.0
                p = t * (_C1 + t * (_C2 + t * _C3))
                z = lax.shift_right_logical(sbm - h, 31)
                af, ae, ac, az = af + p, ae + e, ac + h, az + z
            return af, ae, ac, az

        return lax.fori_loop(0, _D // _L, _iter, carry, unroll=2)

    def _outer(gg, carry):
        for b in (0, 1):
            g = gg * 2 + b
            _wait(b)
            carry = _chunk_acc(b, carry)

            @pl.when(g + 2 < nchunks)
            def _():
                _start(g + 2, b)
        return carry

    zf = jnp.zeros((_L,), jnp.float32)
    zi = jnp.zeros((_L,), jnp.int32)
    accf, acce, cnt, zcnt = lax.fori_loop(0, nchunks // 2, _outer,
                                          (zf, zi, zi, zi))

    stage_f[...] = accf
    stage_e[...] = acce
    stage_c[...] = cnt
    stage_z[...] = zcnt
    pltpu.sync_copy(stage_f, out_f.at[wid])
    pltpu.sync_copy(stage_e, out_e.at[wid])
    pltpu.sync_copy(stage_c, out_c.at[wid])
    pltpu.sync_copy(stage_z, out_z.at[wid])


def _tc_body(hard_ref, soft_ref, logsum_ref, cnt_ref):
    i = pl.program_id(0)
    hard = hard_ref[...]
    mask = hard == 1
    ls = jnp.sum(jnp.where(mask, jnp.log(soft_ref[...]), 0.0))
    c = jnp.sum(hard)

    @pl.when(i == 0)
    def _init():
        logsum_ref[0, 0] = ls
        cnt_ref[0, 0] = c

    @pl.when(i != 0)
    def _acc():
        logsum_ref[0, 0] += ls
        cnt_ref[0, 0] += c


def kernel(hard_attention, soft_attention):
    b, s, d = hard_attention.shape
    rows = b * s
    hard2 = hard_attention.reshape(rows, d)
    soft2 = soft_attention.reshape(rows, d)

    sc_rows = _SC_BATCHES * s
    sc_row0 = rows - sc_rows
    nchunks = sc_rows // (_NW * _RB)

    mesh = plsc.VectorSubcoreMesh(core_axis_name="c", subcore_axis_name="s")
    sc = pl.kernel(
        functools.partial(_sc_body, sc_row0, nchunks),
        out_type=[
            jax.ShapeDtypeStruct((_NW, _L), jnp.float32),
            jax.ShapeDtypeStruct((_NW, _L), jnp.int32),
            jax.ShapeDtypeStruct((_NW, _L), jnp.int32),
            jax.ShapeDtypeStruct((_NW, _L), jnp.int32),
        ],
        mesh=mesh,
        compiler_params=pltpu.CompilerParams(use_tc_tiling_on_sc=True),
        scratch_types=[
            pltpu.VMEM((2, _RB, _D), jnp.int32),
            pltpu.VMEM((2, _RB, _D), jnp.float32),
            pltpu.VMEM((_L,), jnp.float32),
            pltpu.VMEM((_L,), jnp.int32),
            pltpu.VMEM((_L,), jnp.int32),
            pltpu.VMEM((_L,), jnp.int32),
            pltpu.SemaphoreType.DMA,
            pltpu.SemaphoreType.DMA,
        ],
    )
    out_f, out_e, out_c, out_z = sc(hard2, soft2)

    tc_blocks = b - _SC_BATCHES
    ls_tc, cnt_tc = pl.pallas_call(
        _tc_body,
        grid=(tc_blocks,),
        in_specs=[
            pl.BlockSpec((s, d), lambda i: (i, 0)),
            pl.BlockSpec((s, d), lambda i: (i, 0)),
        ],
        out_specs=[
            pl.BlockSpec(memory_space=pltpu.SMEM),
            pl.BlockSpec(memory_space=pltpu.SMEM),
        ],
        out_shape=[
            jax.ShapeDtypeStruct((1, 1), jnp.float32),
            jax.ShapeDtypeStruct((1, 1), jnp.int32),
        ],
    )(hard2, soft2)

    sum_f = jnp.sum(out_f)
    sum_e = jnp.sum(out_e.astype(jnp.float32))
    sum_c = jnp.sum(out_c.astype(jnp.float32))
    any_zero = jnp.sum(out_z) > 0
    ln2 = 0.6931471805599453
    ls_sc = ln2 * (sum_e - 127.0 * sum_c + sum_f)
    ls_sc = jnp.where(any_zero, -jnp.inf, ls_sc)
    log_sum = ls_sc + ls_tc[0, 0]
    count = sum_c + cnt_tc[0, 0].astype(jnp.float32)
    return -log_sum / count


# speedup vs baseline: 6.3591x; 1.5519x over previous
import jax
import jax.numpy as jnp
from jax.experimental import pallas as pl
from jax.experimental.pallas import tpu as pltpu


def _abloss_body(hard_ref, soft_ref, logsum_ref, cnt_ref):
    i = pl.program_id(0)
    hard = hard_ref[...]
    soft = soft_ref[...]
    mask = hard == 1
    ls = jnp.sum(jnp.where(mask, jnp.log(soft), 0.0))
    c = jnp.sum(hard)

    @pl.when(i == 0)
    def _init():
        logsum_ref[0, 0] = ls
        cnt_ref[0, 0] = c

    @pl.when(i != 0)
    def _acc():
        logsum_ref[0, 0] += ls
        cnt_ref[0, 0] += c


def kernel(hard_attention, soft_attention):
    B, S, D = hard_attention.shape
    grid = (B,)
    logsum, cnt = pl.pallas_call(
        _abloss_body,
        grid=grid,
        in_specs=[
            pl.BlockSpec((1, S, D), lambda i: (i, 0, 0)),
            pl.BlockSpec((1, S, D), lambda i: (i, 0, 0)),
        ],
        out_specs=[
            pl.BlockSpec(memory_space=pltpu.SMEM),
            pl.BlockSpec(memory_space=pltpu.SMEM),
        ],
        out_shape=[
            jax.ShapeDtypeStruct((1, 1), jnp.float32),
            jax.ShapeDtypeStruct((1, 1), jnp.int32),
        ],
    )(hard_attention, soft_attention)
    return -logsum[0, 0] / cnt[0, 0].astype(jnp.float32)
